# baseline (speedup 1.0000x reference)
import functools

import jax
import jax.numpy as jnp
from jax import lax
from jax.experimental import pallas as pl
from jax.experimental.pallas import tpu as pltpu
from jax.experimental.pallas import tpu_sc as plsc

B = 1024
L = 20
H = 128
S = 12
LEAKY = 0.2

NC, NS = 2, 16
NW = NC * NS
N_IN = B * L
PW = N_IN // NW
CH = 128
NCH_A = PW // CH
N_NBR = N_IN * S
RW = N_NBR // NW
NCH_B = RW // CH

def _wid():
    return lax.axis_index("s") * NC + lax.axis_index("c")


def _sc_gather_small_body(emb, idx_in3, idx_item3, nbrtab,
                          h_out, item_out, nbr_out,
                          idxv, rowbuf, nbrbuf, sem):
    wid = _wid()
    base = wid * PW
    pltpu.sync_copy(idx_in3.at[wid], idxv)
    for c in range(NCH_A):
        pltpu.async_copy(emb.at[idxv.at[c]], rowbuf, sem).wait()
        pltpu.sync_copy(rowbuf, h_out.at[pl.ds(base + c * CH, CH)])
        pltpu.async_copy(nbrtab.at[idxv.at[c]], nbrbuf, sem).wait()
        pltpu.sync_copy(nbrbuf, nbr_out.at[pl.ds(base + c * CH, CH)])
    pltpu.sync_copy(idx_item3.at[wid], idxv)
    for c in range(NCH_A):
        pltpu.async_copy(emb.at[idxv.at[c]], rowbuf, sem).wait()
        pltpu.sync_copy(rowbuf, item_out.at[pl.ds(base + c * CH, CH)])


def _sc_gather_nbr_body(emb, idx3, out, idxv, buf, sem):
    wid = _wid()
    base = wid * RW
    pltpu.sync_copy(idx3.at[wid], idxv)

    @pl.loop(0, NCH_B)
    def _chunk(c):
        pltpu.async_copy(emb.at[idxv.at[c]], buf, sem).wait()
        pltpu.sync_copy(buf, out.at[pl.ds(base + c * CH, CH)])


@functools.cache
def _sc_kernels():
    mesh = plsc.VectorSubcoreMesh(core_axis_name="c", subcore_axis_name="s",
                                  num_cores=NC, num_subcores=NS)
    gather_small = pl.kernel(
        _sc_gather_small_body,
        out_type=(
            jax.ShapeDtypeStruct((N_IN, H // 2), jnp.int32),
            jax.ShapeDtypeStruct((N_IN, H // 2), jnp.int32),
            jax.ShapeDtypeStruct((N_IN, 32), jnp.int32),
        ),
        mesh=mesh,
        scratch_types=(
            pltpu.VMEM((NCH_A, CH), jnp.int32),
            pltpu.VMEM((CH, H // 2), jnp.int32),
            pltpu.VMEM((CH, 32), jnp.int32),
            pltpu.SemaphoreType.DMA,
        ),
        compiler_params=pltpu.CompilerParams(use_tc_tiling_on_sc=False),
    )
    gather_nbr = pl.kernel(
        _sc_gather_nbr_body,
        out_type=jax.ShapeDtypeStruct((N_NBR, H // 2), jnp.int32),
        mesh=mesh,
        scratch_types=(
            pltpu.VMEM((NCH_B, CH), jnp.int32),
            pltpu.VMEM((CH, H // 2), jnp.int32),
            pltpu.SemaphoreType.DMA,
        ),
        compiler_params=pltpu.CompilerParams(use_tc_tiling_on_sc=False),
    )
    return gather_small, gather_nbr


BB = 8
GRID = B // BB


def _tc_body(h_ref, item_ref, maskf_ref, adj_ref, w_ref, nvt_ref,
             a4_ref, w1a_ref, w1l_ref, w2r_ref, w3_ref, bias_ref,
             out_ref):
    f32, bf16 = jnp.float32, jnp.bfloat16
    a4 = a4_ref[...]
    neg = jnp.full((L, L), -9e15, f32)

    maskf = maskf_ref[...]
    mtile = jnp.concatenate([maskf] * BB, axis=1)
    col_b = lax.broadcasted_iota(jnp.int32, (BB, BB * L), 1) // L
    row_b = lax.broadcasted_iota(jnp.int32, (BB, BB * L), 0)
    mblk = jnp.where(col_b == row_b, mtile, 0.0)
    item_flat = item_ref[...].reshape(BB * L, H)
    ssum = jnp.dot(mblk.astype(bf16), item_flat,
                   preferred_element_type=f32)
    sess_bb = ssum / jnp.sum(maskf, axis=1, keepdims=True)
    sess = jnp.broadcast_to(sess_bb[:, None, :], (BB, L, H)).reshape(BB * L, H)

    local_rows = []
    for b in range(BB):
        hbb = h_ref[b]
        p = (a4[:, None, :] * hbb[None, :, :]).reshape(4 * L, H)
        e = lax.dot_general(p, hbb,
                            (((1,), (1,)), ((), ())),
                            preferred_element_type=f32)
        e = jnp.where(e >= 0, e, LEAKY * e).reshape(4, L, L)
        adjb = adj_ref[b]
        al = jnp.where(adjb == 1, e[0], neg)
        al = jnp.where(adjb == 2, e[1], al)
        al = jnp.where(adjb == 3, e[2], al)
        al = jnp.where(adjb == 4, e[3], al)
        al = al - jnp.max(al, axis=-1, keepdims=True)
        al = jnp.exp(al)
        al = al / jnp.sum(al, axis=-1, keepdims=True)
        local_rows.append(jnp.dot(al.astype(bf16), hbb,
                                  preferred_element_type=f32))
    h_loc = jnp.concatenate(local_rows, axis=0)
    hflat = h_ref[...].reshape(BB * L, H)

    w1l = w1l_ref[...]
    w2r = w2r_ref[...]
    nv_all = nvt_ref[...]
    prod = (sess.astype(bf16)[None, :, :] * nv_all).reshape(S * BB * L, H)
    g_all = jnp.dot(prod, w1a_ref[...],
                    preferred_element_type=f32)
    g3 = g_all.reshape(S, BB * L, H)
    scores = []
    for j in range(S):
        g = g3[j] + w_ref[:, j:j + 1] * w1l
        g = jnp.where(g >= 0, g, LEAKY * g)
        scores.append(jnp.sum(g * w2r, axis=-1, keepdims=True))
    m = scores[0]
    for j in range(1, S):
        m = jnp.maximum(m, scores[j])
    exps = [jnp.exp(sc - m) for sc in scores]
    den = exps[0]
    for j in range(1, S):
        den = den + exps[j]
    inv = 1.0 / den
    neigh = (exps[0] * inv) * nv_all[0].astype(f32)
    for j in range(1, S):
        neigh = neigh + (exps[j] * inv) * nv_all[j].astype(f32)
    cat = jnp.concatenate([hflat, neigh.astype(bf16)], axis=1)
    hg = (jnp.dot(cat, w3_ref[...],
                  preferred_element_type=f32) + bias_ref[...])
    hg = jnp.maximum(hg, 0.0)
    out_ref[...] = (h_loc + hg).reshape(BB, L, H)


def _tc_specs():
    in_specs = [
        pl.BlockSpec((BB, L, H), lambda i: (i, 0, 0)),
        pl.BlockSpec((BB, L, H), lambda i: (i, 0, 0)),
        pl.BlockSpec((BB, L), lambda i: (i, 0)),
        pl.BlockSpec((BB, L, L), lambda i: (i, 0, 0)),
        pl.BlockSpec((BB * L, S), lambda i: (i, 0)),
        pl.BlockSpec((S, BB * L, H), lambda i: (0, i, 0)),
        pl.BlockSpec((4, H), lambda i: (0, 0)),
        pl.BlockSpec((H, H), lambda i: (0, 0)),
        pl.BlockSpec((1, H), lambda i: (0, 0)),
        pl.BlockSpec((1, H), lambda i: (0, 0)),
        pl.BlockSpec((2 * H, H), lambda i: (0, 0)),
        pl.BlockSpec((1, H), lambda i: (0, 0)),
    ]
    out_specs = pl.BlockSpec((BB, L, H), lambda i: (i, 0, 0))
    out_shape = jax.ShapeDtypeStruct((B, L, H), jnp.float32)
    return (GRID,), in_specs, out_specs, out_shape


def _tc_call(*args):
    grid, in_specs, out_specs, out_shape = _tc_specs()
    return pl.pallas_call(
        _tc_body,
        grid=grid,
        in_specs=in_specs,
        out_specs=out_specs,
        out_shape=out_shape,
    )(*args)


def kernel(inputs, adj, mask_item, item, embedding, a_0, a_1, a_2, a_3,
           g_w1, g_w2, g_w3, g_bias, adj_all, num):
    gather_small, gather_nbr = _sc_kernels()
    idx_in3 = inputs.astype(jnp.int32).reshape(NW, NCH_A, CH)
    idx_item3 = item.astype(jnp.int32).reshape(NW, NCH_A, CH)
    v = embedding.shape[0]
    emb = lax.bitcast_convert_type(
        embedding.astype(jnp.bfloat16).reshape(v, H // 2, 2), jnp.int32)
    nbrtab = jnp.concatenate(
        [adj_all.astype(jnp.int32),
         lax.bitcast_convert_type(num.astype(jnp.float32), jnp.int32),
         jnp.zeros((adj_all.shape[0], 8), jnp.int32)], axis=1)
    h_rows, item_rows, nbr_packed = gather_small(emb, idx_in3, idx_item3, nbrtab)
    nbr_ids = nbr_packed[:, :S]
    nbr_w = lax.bitcast_convert_type(nbr_packed[:, S:2 * S], jnp.float32)
    idx3 = nbr_ids.T.reshape(NW, NCH_B, CH)
    nv_flat = gather_nbr(emb, idx3)
    h_rows = lax.bitcast_convert_type(h_rows, jnp.bfloat16).reshape(N_IN, H)
    item_rows = lax.bitcast_convert_type(item_rows, jnp.bfloat16).reshape(N_IN, H)
    nvt3 = lax.bitcast_convert_type(nv_flat, jnp.bfloat16).reshape(S, N_IN, H)
    a4 = jnp.concatenate([a_0.T, a_1.T, a_2.T, a_3.T],
                         axis=0).astype(jnp.bfloat16)
    w1a = g_w1[:H].astype(jnp.bfloat16)
    w1l = g_w1[H:]
    w2r = g_w2.T
    w3 = g_w3.astype(jnp.bfloat16)
    bias = g_bias.reshape(1, H)
    maskf = mask_item.astype(jnp.float32)
    return _tc_call(h_rows.reshape(B, L, H), item_rows.reshape(B, L, H),
                    maskf, adj.astype(jnp.int32), nbr_w, nvt3,
                    a4, w1a, w1l, w2r, w3, bias)


# speedup vs baseline: 2.2045x; 2.2045x over previous
import functools

import jax
import jax.numpy as jnp
from jax import lax
from jax.experimental import pallas as pl
from jax.experimental.pallas import tpu as pltpu
from jax.experimental.pallas import tpu_sc as plsc

B = 1024
L = 20
H = 128
S = 12
LEAKY = 0.2

NC, NS = 2, 16
NW = NC * NS
N_IN = B * L
PW = N_IN // NW
CH = 128
NCH_A = PW // CH
N_NBR = N_IN * S
RW = N_NBR // NW
NCH_B = RW // CH

def _wid():
    return lax.axis_index("s") * NC + lax.axis_index("c")


def _sc_gather_small_body(emb, idx_in3, idx_item3, nbrtab,
                          h_out, item_out, nbr_out,
                          idxv, rowbuf, nbrbuf, sem):
    wid = _wid()
    base = wid * PW
    pltpu.sync_copy(idx_in3.at[wid], idxv)
    for c in range(NCH_A):
        pltpu.async_copy(emb.at[idxv.at[c]], rowbuf, sem).wait()
        pltpu.sync_copy(rowbuf, h_out.at[pl.ds(base + c * CH, CH)])
        pltpu.async_copy(nbrtab.at[idxv.at[c]], nbrbuf, sem).wait()
        pltpu.sync_copy(nbrbuf, nbr_out.at[pl.ds(base + c * CH, CH)])
    pltpu.sync_copy(idx_item3.at[wid], idxv)
    for c in range(NCH_A):
        pltpu.async_copy(emb.at[idxv.at[c]], rowbuf, sem).wait()
        pltpu.sync_copy(rowbuf, item_out.at[pl.ds(base + c * CH, CH)])


def _sc_gather_nbr_body(emb, idx3, out, idxv, buf, sem):
    wid = _wid()
    base = wid * RW
    pltpu.sync_copy(idx3.at[wid], idxv)

    @pl.loop(0, NCH_B)
    def _chunk(c):
        pltpu.async_copy(emb.at[idxv.at[c]], buf, sem).wait()
        pltpu.sync_copy(buf, out.at[pl.ds(base + c * CH, CH)])


@functools.cache
def _sc_kernels():
    mesh = plsc.VectorSubcoreMesh(core_axis_name="c", subcore_axis_name="s",
                                  num_cores=NC, num_subcores=NS)
    gather_small = pl.kernel(
        _sc_gather_small_body,
        out_type=(
            jax.ShapeDtypeStruct((N_IN, H), jnp.bfloat16),
            jax.ShapeDtypeStruct((N_IN, H), jnp.bfloat16),
            jax.ShapeDtypeStruct((N_IN, 32), jnp.int32),
        ),
        mesh=mesh,
        scratch_types=(
            pltpu.VMEM((NCH_A, CH), jnp.int32),
            pltpu.VMEM((CH, H), jnp.bfloat16),
            pltpu.VMEM((CH, 32), jnp.int32),
            pltpu.SemaphoreType.DMA,
        ),
        compiler_params=pltpu.CompilerParams(use_tc_tiling_on_sc=False),
    )
    gather_nbr = pl.kernel(
        _sc_gather_nbr_body,
        out_type=jax.ShapeDtypeStruct((N_NBR, H), jnp.bfloat16),
        mesh=mesh,
        scratch_types=(
            pltpu.VMEM((NCH_B, CH), jnp.int32),
            pltpu.VMEM((CH, H), jnp.bfloat16),
            pltpu.SemaphoreType.DMA,
        ),
        compiler_params=pltpu.CompilerParams(use_tc_tiling_on_sc=False),
    )
    return gather_small, gather_nbr


BB = 8
GRID = B // BB


def _tc_body(h_ref, item_ref, maskf_ref, adj_ref, w_ref, nvt_ref,
             a4_ref, w1a_ref, w1l_ref, w2r_ref, w3_ref, bias_ref,
             out_ref):
    f32, bf16 = jnp.float32, jnp.bfloat16
    a4 = a4_ref[...]
    neg = jnp.full((L, L), -9e15, f32)

    maskf = maskf_ref[...]
    mtile = jnp.concatenate([maskf] * BB, axis=1)
    col_b = lax.broadcasted_iota(jnp.int32, (BB, BB * L), 1) // L
    row_b = lax.broadcasted_iota(jnp.int32, (BB, BB * L), 0)
    mblk = jnp.where(col_b == row_b, mtile, 0.0)
    item_flat = item_ref[...].reshape(BB * L, H)
    ssum = jnp.dot(mblk.astype(bf16), item_flat,
                   preferred_element_type=f32)
    sess_bb = ssum / jnp.sum(maskf, axis=1, keepdims=True)
    sess = jnp.broadcast_to(sess_bb[:, None, :], (BB, L, H)).reshape(BB * L, H)

    local_rows = []
    for b in range(BB):
        hbb = h_ref[b]
        p = (a4[:, None, :] * hbb[None, :, :]).reshape(4 * L, H)
        e = lax.dot_general(p, hbb,
                            (((1,), (1,)), ((), ())),
                            preferred_element_type=f32)
        e = jnp.where(e >= 0, e, LEAKY * e).reshape(4, L, L)
        adjb = adj_ref[b]
        al = jnp.where(adjb == 1, e[0], neg)
        al = jnp.where(adjb == 2, e[1], al)
        al = jnp.where(adjb == 3, e[2], al)
        al = jnp.where(adjb == 4, e[3], al)
        al = al - jnp.max(al, axis=-1, keepdims=True)
        al = jnp.exp(al)
        al = al / jnp.sum(al, axis=-1, keepdims=True)
        local_rows.append(jnp.dot(al.astype(bf16), hbb,
                                  preferred_element_type=f32))
    h_loc = jnp.concatenate(local_rows, axis=0)
    hflat = h_ref[...].reshape(BB * L, H)

    w1l = w1l_ref[...]
    w2r = w2r_ref[...]
    nv_all = nvt_ref[...]
    prod = (sess.astype(bf16)[None, :, :] * nv_all).reshape(S * BB * L, H)
    g_all = jnp.dot(prod, w1a_ref[...],
                    preferred_element_type=f32)
    g3 = g_all.reshape(S, BB * L, H)
    scores = []
    for j in range(S):
        g = g3[j] + w_ref[:, j:j + 1] * w1l
        g = jnp.where(g >= 0, g, LEAKY * g)
        scores.append(jnp.sum(g * w2r, axis=-1, keepdims=True))
    m = scores[0]
    for j in range(1, S):
        m = jnp.maximum(m, scores[j])
    exps = [jnp.exp(sc - m) for sc in scores]
    den = exps[0]
    for j in range(1, S):
        den = den + exps[j]
    inv = 1.0 / den
    neigh = (exps[0] * inv) * nv_all[0].astype(f32)
    for j in range(1, S):
        neigh = neigh + (exps[j] * inv) * nv_all[j].astype(f32)
    cat = jnp.concatenate([hflat, neigh.astype(bf16)], axis=1)
    hg = (jnp.dot(cat, w3_ref[...],
                  preferred_element_type=f32) + bias_ref[...])
    hg = jnp.maximum(hg, 0.0)
    out_ref[...] = (h_loc + hg).reshape(BB, L, H)


def _tc_specs():
    in_specs = [
        pl.BlockSpec((BB, L, H), lambda i: (i, 0, 0)),
        pl.BlockSpec((BB, L, H), lambda i: (i, 0, 0)),
        pl.BlockSpec((BB, L), lambda i: (i, 0)),
        pl.BlockSpec((BB, L, L), lambda i: (i, 0, 0)),
        pl.BlockSpec((BB * L, S), lambda i: (i, 0)),
        pl.BlockSpec((S, BB * L, H), lambda i: (0, i, 0)),
        pl.BlockSpec((4, H), lambda i: (0, 0)),
        pl.BlockSpec((H, H), lambda i: (0, 0)),
        pl.BlockSpec((1, H), lambda i: (0, 0)),
        pl.BlockSpec((1, H), lambda i: (0, 0)),
        pl.BlockSpec((2 * H, H), lambda i: (0, 0)),
        pl.BlockSpec((1, H), lambda i: (0, 0)),
    ]
    out_specs = pl.BlockSpec((BB, L, H), lambda i: (i, 0, 0))
    out_shape = jax.ShapeDtypeStruct((B, L, H), jnp.float32)
    return (GRID,), in_specs, out_specs, out_shape


def _tc_call(*args):
    grid, in_specs, out_specs, out_shape = _tc_specs()
    return pl.pallas_call(
        _tc_body,
        grid=grid,
        in_specs=in_specs,
        out_specs=out_specs,
        out_shape=out_shape,
    )(*args)


def kernel(inputs, adj, mask_item, item, embedding, a_0, a_1, a_2, a_3,
           g_w1, g_w2, g_w3, g_bias, adj_all, num):
    gather_small, gather_nbr = _sc_kernels()
    idx_in3 = inputs.astype(jnp.int32).reshape(NW, NCH_A, CH)
    idx_item3 = item.astype(jnp.int32).reshape(NW, NCH_A, CH)
    emb = embedding.astype(jnp.bfloat16)
    nbrtab = jnp.concatenate(
        [adj_all.astype(jnp.int32),
         lax.bitcast_convert_type(num.astype(jnp.float32), jnp.int32),
         jnp.zeros((adj_all.shape[0], 8), jnp.int32)], axis=1)
    h_rows, item_rows, nbr_packed = gather_small(emb, idx_in3, idx_item3, nbrtab)
    nbr_ids = nbr_packed[:, :S]
    nbr_w = lax.bitcast_convert_type(nbr_packed[:, S:2 * S], jnp.float32)
    idx3 = nbr_ids.T.reshape(NW, NCH_B, CH)
    nv_flat = gather_nbr(emb, idx3)
    nvt3 = nv_flat.reshape(S, N_IN, H)
    a4 = jnp.concatenate([a_0.T, a_1.T, a_2.T, a_3.T],
                         axis=0).astype(jnp.bfloat16)
    w1a = g_w1[:H].astype(jnp.bfloat16)
    w1l = g_w1[H:]
    w2r = g_w2.T
    w3 = g_w3.astype(jnp.bfloat16)
    bias = g_bias.reshape(1, H)
    maskf = mask_item.astype(jnp.float32)
    return _tc_call(h_rows.reshape(B, L, H), item_rows.reshape(B, L, H),
                    maskf, adj.astype(jnp.int32), nbr_w, nvt3,
                    a4, w1a, w1l, w2r, w3, bias)


# speedup vs baseline: 3.2915x; 1.4931x over previous
import functools

import jax
import jax.numpy as jnp
from jax import lax
from jax.experimental import pallas as pl
from jax.experimental.pallas import tpu as pltpu
from jax.experimental.pallas import tpu_sc as plsc

B = 1024
L = 20
H = 128
S = 12
LEAKY = 0.2

NC, NS = 2, 16
NW = NC * NS
N_IN = B * L
PW = N_IN // NW
CH = 128
NCH_A = PW // CH
N_NBR = N_IN * S
RW = N_NBR // NW
NCH_B = RW // CH

def _wid():
    return lax.axis_index("s") * NC + lax.axis_index("c")


def _sc_gather_small_body(emb, idx_in3, idx_item3, nbrtab,
                          h_out, item_out, nbr_out,
                          idxv, rowbuf, nbrbuf, sem):
    wid = _wid()
    base = wid * PW
    pltpu.sync_copy(idx_in3.at[wid], idxv)
    for c in range(NCH_A):
        pltpu.async_copy(emb.at[idxv.at[c]], rowbuf, sem).wait()
        pltpu.sync_copy(rowbuf, h_out.at[pl.ds(base + c * CH, CH)])
        pltpu.async_copy(nbrtab.at[idxv.at[c]], nbrbuf, sem).wait()
        pltpu.sync_copy(nbrbuf, nbr_out.at[pl.ds(base + c * CH, CH)])
    pltpu.sync_copy(idx_item3.at[wid], idxv)
    for c in range(NCH_A):
        pltpu.async_copy(emb.at[idxv.at[c]], rowbuf, sem).wait()
        pltpu.sync_copy(rowbuf, item_out.at[pl.ds(base + c * CH, CH)])


def _sc_gather_nbr_body(emb, idx3, out, idxv, buf, sem):
    wid = _wid()
    base = wid * RW
    pltpu.sync_copy(idx3.at[wid], idxv)

    @pl.loop(0, NCH_B)
    def _chunk(c):
        pltpu.async_copy(emb.at[idxv.at[c]], buf, sem).wait()
        pltpu.sync_copy(buf, out.at[pl.ds(base + c * CH, CH)])


@functools.cache
def _sc_kernels():
    mesh = plsc.VectorSubcoreMesh(core_axis_name="c", subcore_axis_name="s",
                                  num_cores=NC, num_subcores=NS)
    gather_small = pl.kernel(
        _sc_gather_small_body,
        out_type=(
            jax.ShapeDtypeStruct((N_IN, H), jnp.float32),
            jax.ShapeDtypeStruct((N_IN, H), jnp.float32),
            jax.ShapeDtypeStruct((N_IN, 32), jnp.int32),
        ),
        mesh=mesh,
        scratch_types=(
            pltpu.VMEM((NCH_A, CH), jnp.int32),
            pltpu.VMEM((CH, H), jnp.float32),
            pltpu.VMEM((CH, 32), jnp.int32),
            pltpu.SemaphoreType.DMA,
        ),
        compiler_params=pltpu.CompilerParams(use_tc_tiling_on_sc=False),
    )
    gather_nbr = pl.kernel(
        _sc_gather_nbr_body,
        out_type=jax.ShapeDtypeStruct((N_NBR, H), jnp.float32),
        mesh=mesh,
        scratch_types=(
            pltpu.VMEM((NCH_B, CH), jnp.int32),
            pltpu.VMEM((CH, H), jnp.float32),
            pltpu.SemaphoreType.DMA,
        ),
    )
    return gather_small, gather_nbr


BB = 8
GRID = B // BB


def _tc_body(h_ref, item_ref, maskf_ref, adj_ref, w_ref, nvt_ref,
             a4_ref, w1a_ref, w1l_ref, w2r_ref, w3_ref, bias_ref,
             out_ref):
    f32, bf16 = jnp.float32, jnp.bfloat16
    a4 = a4_ref[...]
    neg = jnp.full((L, L), -9e15, f32)

    maskf = maskf_ref[...]
    mtile = jnp.concatenate([maskf] * BB, axis=1)
    col_b = lax.broadcasted_iota(jnp.int32, (BB, BB * L), 1) // L
    row_b = lax.broadcasted_iota(jnp.int32, (BB, BB * L), 0)
    mblk = jnp.where(col_b == row_b, mtile, 0.0)
    item_flat = item_ref[...]
    ssum = jnp.dot(mblk.astype(bf16), item_flat.astype(bf16),
                   preferred_element_type=f32)
    sess_bb = ssum / jnp.sum(maskf, axis=1, keepdims=True)
    sess = jnp.broadcast_to(sess_bb[:, None, :], (BB, L, H)).reshape(BB * L, H)

    local_rows = []
    for b in range(BB):
        hbb = h_ref[b * L:(b + 1) * L, :].astype(bf16)
        p = (a4[:, None, :] * hbb[None, :, :]).reshape(4 * L, H)
        e = lax.dot_general(p, hbb,
                            (((1,), (1,)), ((), ())),
                            preferred_element_type=f32)
        e = jnp.where(e >= 0, e, LEAKY * e).reshape(4, L, L)
        adjb = adj_ref[b * L:(b + 1) * L, :]
        al = jnp.where(adjb == 1, e[0], neg)
        al = jnp.where(adjb == 2, e[1], al)
        al = jnp.where(adjb == 3, e[2], al)
        al = jnp.where(adjb == 4, e[3], al)
        al = al - jnp.max(al, axis=-1, keepdims=True)
        al = jnp.exp(al)
        al = al / jnp.sum(al, axis=-1, keepdims=True)
        local_rows.append(jnp.dot(al.astype(bf16), hbb,
                                  preferred_element_type=f32))
    h_loc = jnp.concatenate(local_rows, axis=0)
    hflat = h_ref[...]

    w1l = w1l_ref[...]
    w2r = w2r_ref[...]
    nv_all = nvt_ref[...]
    prod = (sess[None, :, :] * nv_all).astype(bf16).reshape(S * BB * L, H)
    g_all = jnp.dot(prod, w1a_ref[...],
                    preferred_element_type=f32)
    g3 = g_all.reshape(S, BB * L, H)
    scores = []
    for j in range(S):
        g = g3[j] + w_ref[:, j:j + 1] * w1l
        g = jnp.where(g >= 0, g, LEAKY * g)
        scores.append(jnp.sum(g * w2r, axis=-1, keepdims=True))
    m = scores[0]
    for j in range(1, S):
        m = jnp.maximum(m, scores[j])
    exps = [jnp.exp(sc - m) for sc in scores]
    den = exps[0]
    for j in range(1, S):
        den = den + exps[j]
    inv = 1.0 / den
    neigh = (exps[0] * inv) * nv_all[0]
    for j in range(1, S):
        neigh = neigh + (exps[j] * inv) * nv_all[j]
    cat = jnp.concatenate([hflat, neigh],
                          axis=1).astype(bf16)
    hg = (jnp.dot(cat, w3_ref[...],
                  preferred_element_type=f32) + bias_ref[...])
    hg = jnp.maximum(hg, 0.0)
    out_ref[...] = h_loc + hg


def _tc_specs():
    in_specs = [
        pl.BlockSpec((BB * L, H), lambda i: (i, 0)),
        pl.BlockSpec((BB * L, H), lambda i: (i, 0)),
        pl.BlockSpec((BB, L), lambda i: (i, 0)),
        pl.BlockSpec((BB * L, L), lambda i: (i, 0)),
        pl.BlockSpec((BB * L, S), lambda i: (i, 0)),
        pl.BlockSpec((S, BB * L, H), lambda i: (0, i, 0)),
        pl.BlockSpec((4, H), lambda i: (0, 0)),
        pl.BlockSpec((H, H), lambda i: (0, 0)),
        pl.BlockSpec((1, H), lambda i: (0, 0)),
        pl.BlockSpec((1, H), lambda i: (0, 0)),
        pl.BlockSpec((2 * H, H), lambda i: (0, 0)),
        pl.BlockSpec((1, H), lambda i: (0, 0)),
    ]
    out_specs = pl.BlockSpec((BB * L, H), lambda i: (i, 0))
    out_shape = jax.ShapeDtypeStruct((B * L, H), jnp.float32)
    return (GRID,), in_specs, out_specs, out_shape


def _tc_call(*args):
    grid, in_specs, out_specs, out_shape = _tc_specs()
    return pl.pallas_call(
        _tc_body,
        grid=grid,
        in_specs=in_specs,
        out_specs=out_specs,
        out_shape=out_shape,
    )(*args)


def kernel(inputs, adj, mask_item, item, embedding, a_0, a_1, a_2, a_3,
           g_w1, g_w2, g_w3, g_bias, adj_all, num):
    gather_small, gather_nbr = _sc_kernels()
    idx_in3 = inputs.astype(jnp.int32).reshape(NW, NCH_A, CH)
    idx_item3 = item.astype(jnp.int32).reshape(NW, NCH_A, CH)
    emb = embedding.astype(jnp.float32)
    nbrtab = jnp.concatenate(
        [adj_all.astype(jnp.int32),
         lax.bitcast_convert_type(num.astype(jnp.float32), jnp.int32),
         jnp.zeros((adj_all.shape[0], 8), jnp.int32)], axis=1)
    h_rows, item_rows, nbr_packed = gather_small(emb, idx_in3, idx_item3, nbrtab)
    nbr_ids = nbr_packed[:, :S]
    nbr_w = lax.bitcast_convert_type(nbr_packed[:, S:2 * S], jnp.float32)
    idx3 = nbr_ids.T.reshape(NW, NCH_B, CH)
    nv_flat = gather_nbr(emb, idx3)
    nvt3 = nv_flat.reshape(S, N_IN, H)
    a4 = jnp.concatenate([a_0.T, a_1.T, a_2.T, a_3.T],
                         axis=0).astype(jnp.bfloat16)
    w1a = g_w1[:H].astype(jnp.bfloat16)
    w1l = g_w1[H:]
    w2r = g_w2.T
    w3 = g_w3.astype(jnp.bfloat16)
    bias = g_bias.reshape(1, H)
    maskf = mask_item.astype(jnp.float32)
    adj2 = adj.reshape(N_IN, L)
    out = _tc_call(h_rows, item_rows, maskf, adj2, nbr_w, nvt3,
                   a4, w1a, w1l, w2r, w3, bias)
    return out.reshape(B, L, H)


# speedup vs baseline: 3.4262x; 1.0409x over previous
import functools

import jax
import jax.numpy as jnp
from jax import lax
from jax.experimental import pallas as pl
from jax.experimental.pallas import tpu as pltpu
from jax.experimental.pallas import tpu_sc as plsc

B = 1024
L = 20
H = 128
S = 12
LEAKY = 0.2

NC, NS = 2, 16
NW = NC * NS
N_IN = B * L
PW = N_IN // NW
CH = 128
NCH_A = PW // CH
N_NBR = N_IN * S
RW = N_NBR // NW
NCH_B = RW // CH

def _wid():
    return lax.axis_index("s") * NC + lax.axis_index("c")


def _sc_gather_tab_body(nbrtab, idx3, nbr_out, idxv, nbrbuf, sem):
    wid = _wid()
    base = wid * PW
    pltpu.sync_copy(idx3.at[wid], idxv)
    for c in range(NCH_A):
        pltpu.async_copy(nbrtab.at[idxv.at[c]], nbrbuf, sem).wait()
        pltpu.sync_copy(nbrbuf, nbr_out.at[pl.ds(base + c * CH, CH)])


def _sc_gather_small_body(emb, idx_in3, idx_item3, h_out, item_out,
                          idxv, rowbuf, sem):
    wid = _wid()
    base = wid * PW
    pltpu.sync_copy(idx_in3.at[wid], idxv)
    for c in range(NCH_A):
        pltpu.async_copy(emb.at[idxv.at[c]], rowbuf, sem).wait()
        pltpu.sync_copy(rowbuf, h_out.at[pl.ds(base + c * CH, CH)])
    pltpu.sync_copy(idx_item3.at[wid], idxv)
    for c in range(NCH_A):
        pltpu.async_copy(emb.at[idxv.at[c]], rowbuf, sem).wait()
        pltpu.sync_copy(rowbuf, item_out.at[pl.ds(base + c * CH, CH)])


def _sc_gather_nbr_body(emb, idx3, out, idxv, buf, sem):
    wid = _wid()
    base = wid * RW
    pltpu.sync_copy(idx3.at[wid], idxv)

    @pl.loop(0, NCH_B)
    def _chunk(c):
        pltpu.async_copy(emb.at[idxv.at[c]], buf, sem).wait()
        pltpu.sync_copy(buf, out.at[pl.ds(base + c * CH, CH)])


@functools.cache
def _sc_kernels():
    mesh = plsc.VectorSubcoreMesh(core_axis_name="c", subcore_axis_name="s",
                                  num_cores=NC, num_subcores=NS)
    gather_tab = pl.kernel(
        _sc_gather_tab_body,
        out_type=jax.ShapeDtypeStruct((N_IN, 32), jnp.int32),
        mesh=mesh,
        scratch_types=(
            pltpu.VMEM((NCH_A, CH), jnp.int32),
            pltpu.VMEM((CH, 32), jnp.int32),
            pltpu.SemaphoreType.DMA,
        ),
        compiler_params=pltpu.CompilerParams(use_tc_tiling_on_sc=False),
    )
    gather_small = pl.kernel(
        _sc_gather_small_body,
        out_type=(
            jax.ShapeDtypeStruct((N_IN, H), jnp.float32),
            jax.ShapeDtypeStruct((N_IN, H), jnp.float32),
        ),
        mesh=mesh,
        scratch_types=(
            pltpu.VMEM((NCH_A, CH), jnp.int32),
            pltpu.VMEM((CH, H), jnp.float32),
            pltpu.SemaphoreType.DMA,
        ),
    )
    gather_nbr = pl.kernel(
        _sc_gather_nbr_body,
        out_type=jax.ShapeDtypeStruct((N_NBR, H), jnp.float32),
        mesh=mesh,
        scratch_types=(
            pltpu.VMEM((NCH_B, CH), jnp.int32),
            pltpu.VMEM((CH, H), jnp.float32),
            pltpu.SemaphoreType.DMA,
        ),
    )
    return gather_tab, gather_small, gather_nbr


BB = 8
GRID = B // BB


def _tc_local_body(h_ref, item_ref, maskf_ref, adj_ref, a4_ref,
                   loc_ref, sess_ref):
    f32, bf16 = jnp.float32, jnp.bfloat16
    a4 = a4_ref[...]
    neg = jnp.full((L, L), -9e15, f32)

    maskf = maskf_ref[...]
    mtile = jnp.concatenate([maskf] * BB, axis=1)
    col_b = lax.broadcasted_iota(jnp.int32, (BB, BB * L), 1) // L
    row_b = lax.broadcasted_iota(jnp.int32, (BB, BB * L), 0)
    mblk = jnp.where(col_b == row_b, mtile, 0.0)
    ssum = jnp.dot(mblk.astype(bf16), item_ref[...].astype(bf16),
                   preferred_element_type=f32)
    sess_bb = ssum / jnp.sum(maskf, axis=1, keepdims=True)
    sess = jnp.broadcast_to(sess_bb[:, None, :], (BB, L, H)).reshape(BB * L, H)
    sess_ref[...] = sess.astype(bf16)

    local_rows = []
    for b in range(BB):
        hbb = h_ref[b * L:(b + 1) * L, :].astype(bf16)
        p = (a4[:, None, :] * hbb[None, :, :]).reshape(4 * L, H)
        e = lax.dot_general(p, hbb,
                            (((1,), (1,)), ((), ())),
                            preferred_element_type=f32)
        e = jnp.where(e >= 0, e, LEAKY * e).reshape(4, L, L)
        adjb = adj_ref[b * L:(b + 1) * L, :]
        al = jnp.where(adjb == 1, e[0], neg)
        al = jnp.where(adjb == 2, e[1], al)
        al = jnp.where(adjb == 3, e[2], al)
        al = jnp.where(adjb == 4, e[3], al)
        al = al - jnp.max(al, axis=-1, keepdims=True)
        al = jnp.exp(al)
        al = al / jnp.sum(al, axis=-1, keepdims=True)
        local_rows.append(jnp.dot(al.astype(bf16), hbb,
                                  preferred_element_type=f32))
    loc_ref[...] = jnp.concatenate(local_rows, axis=0).astype(bf16)


def _tc_global_body(h_ref, sess_ref, loc_ref, w_ref, nvt_ref,
                    w1a_ref, w1l_ref, w2r_ref, w3_ref, bias_ref, out_ref):
    f32, bf16 = jnp.float32, jnp.bfloat16
    sess = sess_ref[...]
    w1l = w1l_ref[...]
    w2r = w2r_ref[...]
    nv_all = nvt_ref[...]
    prod = (sess[None, :, :] * nv_all.astype(bf16)).reshape(S * BB * L, H)
    g_all = jnp.dot(prod, w1a_ref[...],
                    preferred_element_type=f32)
    g3 = g_all.reshape(S, BB * L, H)
    scores = []
    for j in range(S):
        g = g3[j] + w_ref[:, j:j + 1] * w1l
        g = jnp.where(g >= 0, g, LEAKY * g)
        scores.append(jnp.sum(g * w2r, axis=-1, keepdims=True))
    m = scores[0]
    for j in range(1, S):
        m = jnp.maximum(m, scores[j])
    exps = [jnp.exp(sc - m) for sc in scores]
    den = exps[0]
    for j in range(1, S):
        den = den + exps[j]
    inv = 1.0 / den
    neigh = (exps[0] * inv) * nv_all[0]
    for j in range(1, S):
        neigh = neigh + (exps[j] * inv) * nv_all[j]
    cat = jnp.concatenate([h_ref[...], neigh],
                          axis=1).astype(bf16)
    hg = (jnp.dot(cat, w3_ref[...],
                  preferred_element_type=f32) + bias_ref[...])
    hg = jnp.maximum(hg, 0.0)
    out_ref[...] = loc_ref[...].astype(f32) + hg


def _tc_local_specs():
    in_specs = [
        pl.BlockSpec((BB * L, H), lambda i: (i, 0)),
        pl.BlockSpec((BB * L, H), lambda i: (i, 0)),
        pl.BlockSpec((BB, L), lambda i: (i, 0)),
        pl.BlockSpec((BB * L, L), lambda i: (i, 0)),
        pl.BlockSpec((4, H), lambda i: (0, 0)),
    ]
    out_specs = (pl.BlockSpec((BB * L, H), lambda i: (i, 0)),
                 pl.BlockSpec((BB * L, H), lambda i: (i, 0)))
    out_shape = (jax.ShapeDtypeStruct((B * L, H), jnp.bfloat16),
                 jax.ShapeDtypeStruct((B * L, H), jnp.bfloat16))
    return (GRID,), in_specs, out_specs, out_shape


def _tc_global_specs():
    in_specs = [
        pl.BlockSpec((BB * L, H), lambda i: (i, 0)),
        pl.BlockSpec((BB * L, H), lambda i: (i, 0)),
        pl.BlockSpec((BB * L, H), lambda i: (i, 0)),
        pl.BlockSpec((BB * L, S), lambda i: (i, 0)),
        pl.BlockSpec((S, BB * L, H), lambda i: (0, i, 0)),
        pl.BlockSpec((H, H), lambda i: (0, 0)),
        pl.BlockSpec((1, H), lambda i: (0, 0)),
        pl.BlockSpec((1, H), lambda i: (0, 0)),
        pl.BlockSpec((2 * H, H), lambda i: (0, 0)),
        pl.BlockSpec((1, H), lambda i: (0, 0)),
    ]
    out_specs = pl.BlockSpec((BB * L, H), lambda i: (i, 0))
    out_shape = jax.ShapeDtypeStruct((B * L, H), jnp.float32)
    return (GRID,), in_specs, out_specs, out_shape


def _tc_local_call(*args):
    grid, in_specs, out_specs, out_shape = _tc_local_specs()
    return pl.pallas_call(_tc_local_body, grid=grid, in_specs=in_specs,
                          out_specs=out_specs, out_shape=out_shape)(*args)


def _tc_global_call(*args):
    grid, in_specs, out_specs, out_shape = _tc_global_specs()
    return pl.pallas_call(_tc_global_body, grid=grid, in_specs=in_specs,
                          out_specs=out_specs, out_shape=out_shape)(*args)


def kernel(inputs, adj, mask_item, item, embedding, a_0, a_1, a_2, a_3,
           g_w1, g_w2, g_w3, g_bias, adj_all, num):
    gather_tab, gather_small, gather_nbr = _sc_kernels()
    idx_in3 = inputs.astype(jnp.int32).reshape(NW, NCH_A, CH)
    idx_item3 = item.astype(jnp.int32).reshape(NW, NCH_A, CH)
    emb = embedding.astype(jnp.float32)
    nbrtab = jnp.concatenate(
        [adj_all.astype(jnp.int32),
         lax.bitcast_convert_type(num.astype(jnp.float32), jnp.int32),
         jnp.zeros((adj_all.shape[0], 8), jnp.int32)], axis=1)
    nbr_packed = gather_tab(nbrtab, idx_in3)
    h_rows, item_rows = gather_small(emb, idx_in3, idx_item3)
    nbr_ids = nbr_packed[:, :S]
    nbr_w = lax.bitcast_convert_type(nbr_packed[:, S:2 * S], jnp.float32)
    idx3 = nbr_ids.T.reshape(NW, NCH_B, CH)
    nv_flat = gather_nbr(emb, idx3)
    nvt3 = nv_flat.reshape(S, N_IN, H)
    a4 = jnp.concatenate([a_0.T, a_1.T, a_2.T, a_3.T],
                         axis=0).astype(jnp.bfloat16)
    w1a = g_w1[:H].astype(jnp.bfloat16)
    w1l = g_w1[H:]
    w2r = g_w2.T
    w3 = g_w3.astype(jnp.bfloat16)
    bias = g_bias.reshape(1, H)
    maskf = mask_item.astype(jnp.float32)
    adj2 = adj.reshape(N_IN, L)
    loc_bf, sess_bf = _tc_local_call(h_rows, item_rows, maskf, adj2, a4)
    out = _tc_global_call(h_rows, sess_bf, loc_bf, nbr_w, nvt3,
                          w1a, w1l, w2r, w3, bias)
    return out.reshape(B, L, H)


# speedup vs baseline: 3.8061x; 1.1109x over previous
import functools

import jax
import jax.numpy as jnp
from jax import lax
from jax.experimental import pallas as pl
from jax.experimental.pallas import tpu as pltpu
from jax.experimental.pallas import tpu_sc as plsc

B = 1024
L = 20
H = 128
S = 12
LEAKY = 0.2

NC, NS = 2, 16
NW = NC * NS
N_IN = B * L
PW = N_IN // NW
CH = 128
NCH_A = PW // CH
N_NBR = N_IN * S
RW = N_NBR // NW
NCH_B = RW // CH

def _wid():
    return lax.axis_index("s") * NC + lax.axis_index("c")


def _sc_gather_tab_body(nbrtab, idx3, nbr_out, idxv, nbrbuf, sem):
    wid = _wid()
    base = wid * PW
    pltpu.sync_copy(idx3.at[wid], idxv)
    for c in range(NCH_A):
        pltpu.async_copy(nbrtab.at[idxv.at[c]], nbrbuf, sem).wait()
        pltpu.sync_copy(nbrbuf, nbr_out.at[pl.ds(base + c * CH, CH)])


def _sc_gather_small_body(emb, idx_in3, idx_item3, h_out, item_out,
                          idxv, rowbuf, sem):
    wid = _wid()
    base = wid * PW
    pltpu.sync_copy(idx_in3.at[wid], idxv)
    for c in range(NCH_A):
        pltpu.async_copy(emb.at[idxv.at[c]], rowbuf, sem).wait()
        pltpu.sync_copy(rowbuf, h_out.at[pl.ds(base + c * CH, CH)])
    pltpu.sync_copy(idx_item3.at[wid], idxv)
    for c in range(NCH_A):
        pltpu.async_copy(emb.at[idxv.at[c]], rowbuf, sem).wait()
        pltpu.sync_copy(rowbuf, item_out.at[pl.ds(base + c * CH, CH)])


def _sc_gather_nbr_body(emb, idx3, out, idxv, buf, sem):
    wid = _wid()
    base = wid * RW
    pltpu.sync_copy(idx3.at[wid], idxv)

    @pl.loop(0, NCH_B)
    def _chunk(c):
        pltpu.async_copy(emb.at[idxv.at[c]], buf, sem).wait()
        pltpu.sync_copy(buf, out.at[pl.ds(base + c * CH, CH)])


@functools.cache
def _sc_kernels():
    mesh = plsc.VectorSubcoreMesh(core_axis_name="c", subcore_axis_name="s",
                                  num_cores=NC, num_subcores=NS)
    gather_tab = pl.kernel(
        _sc_gather_tab_body,
        out_type=jax.ShapeDtypeStruct((N_IN, 32), jnp.int32),
        mesh=mesh,
        scratch_types=(
            pltpu.VMEM((NCH_A, CH), jnp.int32),
            pltpu.VMEM((CH, 32), jnp.int32),
            pltpu.SemaphoreType.DMA,
        ),
        compiler_params=pltpu.CompilerParams(use_tc_tiling_on_sc=False),
    )
    gather_small = pl.kernel(
        _sc_gather_small_body,
        out_type=(
            jax.ShapeDtypeStruct((N_IN, H), jnp.float32),
            jax.ShapeDtypeStruct((N_IN, H), jnp.float32),
        ),
        mesh=mesh,
        scratch_types=(
            pltpu.VMEM((NCH_A, CH), jnp.int32),
            pltpu.VMEM((CH, H), jnp.float32),
            pltpu.SemaphoreType.DMA,
        ),
    )
    gather_nbr = pl.kernel(
        _sc_gather_nbr_body,
        out_type=jax.ShapeDtypeStruct((N_NBR, H), jnp.float32),
        mesh=mesh,
        scratch_types=(
            pltpu.VMEM((NCH_B, CH), jnp.int32),
            pltpu.VMEM((CH, H), jnp.float32),
            pltpu.SemaphoreType.DMA,
        ),
    )
    return gather_tab, gather_small, gather_nbr


BB = 8
GRID = B // BB


def _tc_local_body(h_ref, item_ref, maskf_ref, adj_ref, a4_ref,
                   loc_ref, sess_ref):
    f32, bf16 = jnp.float32, jnp.bfloat16
    a4 = a4_ref[...]

    maskf = maskf_ref[...]
    mtile = jnp.concatenate([maskf] * BB, axis=1)
    col_b = lax.broadcasted_iota(jnp.int32, (BB, BB * L), 1) // L
    row_b = lax.broadcasted_iota(jnp.int32, (BB, BB * L), 0)
    mblk = jnp.where(col_b == row_b, mtile, 0.0)
    ssum = jnp.dot(mblk.astype(bf16), item_ref[...].astype(bf16),
                   preferred_element_type=f32)
    sess_bb = ssum / jnp.sum(maskf, axis=1, keepdims=True)
    sess = jnp.broadcast_to(sess_bb[:, None, :], (BB, L, H)).reshape(BB * L, H)
    sess_ref[...] = sess.astype(bf16)

    hb_all = h_ref[...].astype(bf16)
    N = BB * L
    adj_tile = jnp.concatenate([adj_ref[...]] * BB, axis=1)
    colb = lax.broadcasted_iota(jnp.int32, (N, N), 1) // L
    rowb = lax.broadcasted_iota(jnp.int32, (N, N), 0) // L
    in_blk = colb == rowb
    es = []
    for k in range(4):
        ek = lax.dot_general(hb_all * a4[k:k + 1, :], hb_all,
                             (((1,), (1,)), ((), ())),
                             preferred_element_type=f32)
        es.append(jnp.where(ek >= 0, ek, LEAKY * ek))
    neg_in = jnp.full((N, N), -9e15, f32)
    al = jnp.where(adj_tile == 1, es[0], neg_in)
    al = jnp.where(adj_tile == 2, es[1], al)
    al = jnp.where(adj_tile == 3, es[2], al)
    al = jnp.where(adj_tile == 4, es[3], al)
    al = jnp.where(in_blk, al, -1.8e16)
    al = al - jnp.max(al, axis=-1, keepdims=True)
    al = jnp.exp(al)
    al = al / jnp.sum(al, axis=-1, keepdims=True)
    loc = jnp.dot(al.astype(bf16), hb_all, preferred_element_type=f32)
    loc_ref[...] = loc.astype(bf16)


def _tc_global_body(h_ref, sess_ref, loc_ref, w_ref, nvt_ref,
                    w1a_ref, w1l_ref, w2r_ref, w3_ref, bias_ref, out_ref):
    f32, bf16 = jnp.float32, jnp.bfloat16
    sess = sess_ref[...]
    w1l = w1l_ref[...]
    w2r = w2r_ref[...]
    nv_all = nvt_ref[...]
    prod = (sess[None, :, :] * nv_all.astype(bf16)).reshape(S * BB * L, H)
    g_all = jnp.dot(prod, w1a_ref[...],
                    preferred_element_type=f32)
    g3 = g_all.reshape(S, BB * L, H)
    scores = []
    for j in range(S):
        g = g3[j] + w_ref[:, j:j + 1] * w1l
        g = jnp.where(g >= 0, g, LEAKY * g)
        scores.append(jnp.sum(g * w2r, axis=-1, keepdims=True))
    m = scores[0]
    for j in range(1, S):
        m = jnp.maximum(m, scores[j])
    exps = [jnp.exp(sc - m) for sc in scores]
    den = exps[0]
    for j in range(1, S):
        den = den + exps[j]
    inv = 1.0 / den
    neigh = (exps[0] * inv) * nv_all[0]
    for j in range(1, S):
        neigh = neigh + (exps[j] * inv) * nv_all[j]
    cat = jnp.concatenate([h_ref[...], neigh],
                          axis=1).astype(bf16)
    hg = (jnp.dot(cat, w3_ref[...],
                  preferred_element_type=f32) + bias_ref[...])
    hg = jnp.maximum(hg, 0.0)
    out_ref[...] = loc_ref[...].astype(f32) + hg


def _tc_local_specs():
    in_specs = [
        pl.BlockSpec((BB * L, H), lambda i: (i, 0)),
        pl.BlockSpec((BB * L, H), lambda i: (i, 0)),
        pl.BlockSpec((BB, L), lambda i: (i, 0)),
        pl.BlockSpec((BB * L, L), lambda i: (i, 0)),
        pl.BlockSpec((4, H), lambda i: (0, 0)),
    ]
    out_specs = (pl.BlockSpec((BB * L, H), lambda i: (i, 0)),
                 pl.BlockSpec((BB * L, H), lambda i: (i, 0)))
    out_shape = (jax.ShapeDtypeStruct((B * L, H), jnp.bfloat16),
                 jax.ShapeDtypeStruct((B * L, H), jnp.bfloat16))
    return (GRID,), in_specs, out_specs, out_shape


def _tc_global_specs():
    in_specs = [
        pl.BlockSpec((BB * L, H), lambda i: (i, 0)),
        pl.BlockSpec((BB * L, H), lambda i: (i, 0)),
        pl.BlockSpec((BB * L, H), lambda i: (i, 0)),
        pl.BlockSpec((BB * L, S), lambda i: (i, 0)),
        pl.BlockSpec((S, BB * L, H), lambda i: (0, i, 0)),
        pl.BlockSpec((H, H), lambda i: (0, 0)),
        pl.BlockSpec((1, H), lambda i: (0, 0)),
        pl.BlockSpec((1, H), lambda i: (0, 0)),
        pl.BlockSpec((2 * H, H), lambda i: (0, 0)),
        pl.BlockSpec((1, H), lambda i: (0, 0)),
    ]
    out_specs = pl.BlockSpec((BB * L, H), lambda i: (i, 0))
    out_shape = jax.ShapeDtypeStruct((B * L, H), jnp.float32)
    return (GRID,), in_specs, out_specs, out_shape


def _tc_local_call(*args):
    grid, in_specs, out_specs, out_shape = _tc_local_specs()
    return pl.pallas_call(_tc_local_body, grid=grid, in_specs=in_specs,
                          out_specs=out_specs, out_shape=out_shape)(*args)


def _tc_global_call(*args):
    grid, in_specs, out_specs, out_shape = _tc_global_specs()
    return pl.pallas_call(_tc_global_body, grid=grid, in_specs=in_specs,
                          out_specs=out_specs, out_shape=out_shape)(*args)


def kernel(inputs, adj, mask_item, item, embedding, a_0, a_1, a_2, a_3,
           g_w1, g_w2, g_w3, g_bias, adj_all, num):
    gather_tab, gather_small, gather_nbr = _sc_kernels()
    idx_in3 = inputs.astype(jnp.int32).reshape(NW, NCH_A, CH)
    idx_item3 = item.astype(jnp.int32).reshape(NW, NCH_A, CH)
    emb = embedding
    nbrtab = jnp.concatenate(
        [adj_all, lax.bitcast_convert_type(num, jnp.int32),
         jnp.zeros((adj_all.shape[0], 8), jnp.int32)], axis=1)
    nbr_packed = gather_tab(nbrtab, idx_in3)
    h_rows, item_rows = gather_small(emb, idx_in3, idx_item3)
    nbr_ids = nbr_packed[:, :S]
    nbr_w = lax.bitcast_convert_type(nbr_packed[:, S:2 * S], jnp.float32)
    idx3 = nbr_ids.T.reshape(NW, NCH_B, CH)
    nv_flat = gather_nbr(emb, idx3)
    nvt3 = nv_flat.reshape(S, N_IN, H)
    a4 = jnp.concatenate([a_0.T, a_1.T, a_2.T, a_3.T],
                         axis=0).astype(jnp.bfloat16)
    w1a = g_w1[:H].astype(jnp.bfloat16)
    w1l = g_w1[H:]
    w2r = g_w2.T
    w3 = g_w3.astype(jnp.bfloat16)
    bias = g_bias.reshape(1, H)
    maskf = mask_item.astype(jnp.float32)
    adj2 = adj.reshape(N_IN, L)
    loc_bf, sess_bf = _tc_local_call(h_rows, item_rows, maskf, adj2, a4)
    out = _tc_global_call(h_rows, sess_bf, loc_bf, nbr_w, nvt3,
                          w1a, w1l, w2r, w3, bias)
    return out.reshape(B, L, H)


# speedup vs baseline: 4.4834x; 1.1780x over previous
import functools

import jax
import jax.numpy as jnp
from jax import lax
from jax.experimental import pallas as pl
from jax.experimental.pallas import tpu as pltpu
from jax.experimental.pallas import tpu_sc as plsc

B = 1024
L = 20
H = 128
S = 12
LEAKY = 0.2

NC, NS = 2, 16
NW = NC * NS
N_IN = B * L
PW = N_IN // NW
CH = 128
NCH_A = PW // CH
N_NBR = N_IN * S
RW = N_NBR // NW
NCH_B = RW // CH

def _wid():
    return lax.axis_index("s") * NC + lax.axis_index("c")


def _sc_gather_tab_body(nbrtab, idx3, nbr_out, idxv, nbrbuf, sem):
    wid = _wid()
    base = wid * PW
    pltpu.sync_copy(idx3.at[wid], idxv)
    for c in range(NCH_A):
        pltpu.async_copy(nbrtab.at[idxv.at[c]], nbrbuf, sem).wait()
        pltpu.sync_copy(nbrbuf, nbr_out.at[pl.ds(base + c * CH, CH)])


def _sc_gather_small_body(emb, idx_in3, idx_item3, h_out, item_out,
                          idxv, rowbuf, sem):
    wid = _wid()
    base = wid * PW
    pltpu.sync_copy(idx_in3.at[wid], idxv)
    for c in range(NCH_A):
        pltpu.async_copy(emb.at[idxv.at[c]], rowbuf, sem).wait()
        pltpu.sync_copy(rowbuf, h_out.at[pl.ds(base + c * CH, CH)])
    pltpu.sync_copy(idx_item3.at[wid], idxv)
    for c in range(NCH_A):
        pltpu.async_copy(emb.at[idxv.at[c]], rowbuf, sem).wait()
        pltpu.sync_copy(rowbuf, item_out.at[pl.ds(base + c * CH, CH)])


def _sc_gather_nbr_body(emb, idx3, out, idxv, buf, sem):
    wid = _wid()
    base = wid * RW
    pltpu.sync_copy(idx3.at[wid], idxv)

    @pl.loop(0, NCH_B)
    def _chunk(c):
        pltpu.async_copy(emb.at[idxv.at[c]], buf, sem).wait()
        pltpu.sync_copy(buf, out.at[pl.ds(base + c * CH, CH)])


@functools.cache
def _sc_kernels():
    mesh = plsc.VectorSubcoreMesh(core_axis_name="c", subcore_axis_name="s",
                                  num_cores=NC, num_subcores=NS)
    gather_tab = pl.kernel(
        _sc_gather_tab_body,
        out_type=jax.ShapeDtypeStruct((N_IN, 32), jnp.int32),
        mesh=mesh,
        scratch_types=(
            pltpu.VMEM((NCH_A, CH), jnp.int32),
            pltpu.VMEM((CH, 32), jnp.int32),
            pltpu.SemaphoreType.DMA,
        ),
        compiler_params=pltpu.CompilerParams(use_tc_tiling_on_sc=False),
    )
    gather_small = pl.kernel(
        _sc_gather_small_body,
        out_type=(
            jax.ShapeDtypeStruct((N_IN, H), jnp.float32),
            jax.ShapeDtypeStruct((N_IN, H), jnp.float32),
        ),
        mesh=mesh,
        scratch_types=(
            pltpu.VMEM((NCH_A, CH), jnp.int32),
            pltpu.VMEM((CH, H), jnp.float32),
            pltpu.SemaphoreType.DMA,
        ),
    )
    gather_nbr = pl.kernel(
        _sc_gather_nbr_body,
        out_type=jax.ShapeDtypeStruct((N_NBR, H), jnp.float32),
        mesh=mesh,
        scratch_types=(
            pltpu.VMEM((NCH_B, CH), jnp.int32),
            pltpu.VMEM((CH, H), jnp.float32),
            pltpu.SemaphoreType.DMA,
        ),
    )
    return gather_tab, gather_small, gather_nbr


BB = 16
GRID = B // BB


def _tc_local_body(h_ref, item_ref, maskf_ref, adj_ref, a4_ref,
                   loc_ref, sess_ref):
    f32, bf16 = jnp.float32, jnp.bfloat16
    a4 = a4_ref[...]

    maskf = maskf_ref[...]
    mtile = jnp.concatenate([maskf] * BB, axis=1)
    col_b = lax.broadcasted_iota(jnp.int32, (BB, BB * L), 1) // L
    row_b = lax.broadcasted_iota(jnp.int32, (BB, BB * L), 0)
    mblk = jnp.where(col_b == row_b, mtile, 0.0)
    ssum = jnp.dot(mblk.astype(bf16), item_ref[...].astype(bf16),
                   preferred_element_type=f32)
    sess_bb = ssum / jnp.sum(maskf, axis=1, keepdims=True)
    sess = jnp.broadcast_to(sess_bb[:, None, :], (BB, L, H)).reshape(BB * L, H)
    sess_ref[...] = sess.astype(bf16)

    hb_all = h_ref[...].astype(bf16)
    N = BB * L
    adj_tile = jnp.concatenate([adj_ref[...]] * BB, axis=1)
    colb = lax.broadcasted_iota(jnp.int32, (N, N), 1) // L
    rowb = lax.broadcasted_iota(jnp.int32, (N, N), 0) // L
    in_blk = colb == rowb
    es = []
    for k in range(4):
        ek = lax.dot_general(hb_all * a4[k:k + 1, :], hb_all,
                             (((1,), (1,)), ((), ())),
                             preferred_element_type=f32)
        es.append(jnp.where(ek >= 0, ek, LEAKY * ek))
    neg_in = jnp.full((N, N), -9e15, f32)
    al = jnp.where(adj_tile == 1, es[0], neg_in)
    al = jnp.where(adj_tile == 2, es[1], al)
    al = jnp.where(adj_tile == 3, es[2], al)
    al = jnp.where(adj_tile == 4, es[3], al)
    al = jnp.where(in_blk, al, -1.8e16)
    al = al - jnp.max(al, axis=-1, keepdims=True)
    al = jnp.exp(al)
    al = al / jnp.sum(al, axis=-1, keepdims=True)
    loc = jnp.dot(al.astype(bf16), hb_all, preferred_element_type=f32)
    loc_ref[...] = loc.astype(bf16)


def _tc_global_body(h_ref, sess_ref, loc_ref, w_ref, nvt_ref,
                    w1a_ref, w1l_ref, w2r_ref, w3_ref, bias_ref, out_ref):
    f32, bf16 = jnp.float32, jnp.bfloat16
    sess = sess_ref[...]
    w1l = w1l_ref[...]
    w2r = w2r_ref[...]
    nv_all = nvt_ref[...]
    prod = (sess[None, :, :] * nv_all.astype(bf16)).reshape(S * BB * L, H)
    g_all = jnp.dot(prod, w1a_ref[...],
                    preferred_element_type=f32)
    g3 = g_all.reshape(S, BB * L, H)
    scores = []
    for j in range(S):
        g = g3[j] + w_ref[:, j:j + 1] * w1l
        g = jnp.where(g >= 0, g, LEAKY * g)
        scores.append(jnp.sum(g * w2r, axis=-1, keepdims=True))
    m = scores[0]
    for j in range(1, S):
        m = jnp.maximum(m, scores[j])
    exps = [jnp.exp(sc - m) for sc in scores]
    den = exps[0]
    for j in range(1, S):
        den = den + exps[j]
    inv = 1.0 / den
    neigh = (exps[0] * inv) * nv_all[0]
    for j in range(1, S):
        neigh = neigh + (exps[j] * inv) * nv_all[j]
    cat = jnp.concatenate([h_ref[...], neigh],
                          axis=1).astype(bf16)
    hg = (jnp.dot(cat, w3_ref[...],
                  preferred_element_type=f32) + bias_ref[...])
    hg = jnp.maximum(hg, 0.0)
    out_ref[...] = loc_ref[...].astype(f32) + hg


def _tc_local_specs():
    in_specs = [
        pl.BlockSpec((BB * L, H), lambda i: (i, 0)),
        pl.BlockSpec((BB * L, H), lambda i: (i, 0)),
        pl.BlockSpec((BB, L), lambda i: (i, 0)),
        pl.BlockSpec((BB * L, L), lambda i: (i, 0)),
        pl.BlockSpec((4, H), lambda i: (0, 0)),
    ]
    out_specs = (pl.BlockSpec((BB * L, H), lambda i: (i, 0)),
                 pl.BlockSpec((BB * L, H), lambda i: (i, 0)))
    out_shape = (jax.ShapeDtypeStruct((B * L, H), jnp.bfloat16),
                 jax.ShapeDtypeStruct((B * L, H), jnp.bfloat16))
    return (GRID,), in_specs, out_specs, out_shape


def _tc_global_specs():
    in_specs = [
        pl.BlockSpec((BB * L, H), lambda i: (i, 0)),
        pl.BlockSpec((BB * L, H), lambda i: (i, 0)),
        pl.BlockSpec((BB * L, H), lambda i: (i, 0)),
        pl.BlockSpec((BB * L, S), lambda i: (i, 0)),
        pl.BlockSpec((S, BB * L, H), lambda i: (0, i, 0)),
        pl.BlockSpec((H, H), lambda i: (0, 0)),
        pl.BlockSpec((1, H), lambda i: (0, 0)),
        pl.BlockSpec((1, H), lambda i: (0, 0)),
        pl.BlockSpec((2 * H, H), lambda i: (0, 0)),
        pl.BlockSpec((1, H), lambda i: (0, 0)),
    ]
    out_specs = pl.BlockSpec((BB * L, H), lambda i: (i, 0))
    out_shape = jax.ShapeDtypeStruct((B * L, H), jnp.float32)
    return (GRID,), in_specs, out_specs, out_shape


def _tc_local_call(*args):
    grid, in_specs, out_specs, out_shape = _tc_local_specs()
    return pl.pallas_call(_tc_local_body, grid=grid, in_specs=in_specs,
                          out_specs=out_specs, out_shape=out_shape)(*args)


def _tc_global_call(*args):
    grid, in_specs, out_specs, out_shape = _tc_global_specs()
    return pl.pallas_call(_tc_global_body, grid=grid, in_specs=in_specs,
                          out_specs=out_specs, out_shape=out_shape)(*args)


def kernel(inputs, adj, mask_item, item, embedding, a_0, a_1, a_2, a_3,
           g_w1, g_w2, g_w3, g_bias, adj_all, num):
    gather_tab, gather_small, gather_nbr = _sc_kernels()
    idx_in3 = inputs.astype(jnp.int32).reshape(NW, NCH_A, CH)
    idx_item3 = item.astype(jnp.int32).reshape(NW, NCH_A, CH)
    emb = embedding
    nbrtab = jnp.concatenate(
        [adj_all, lax.bitcast_convert_type(num, jnp.int32),
         jnp.zeros((adj_all.shape[0], 8), jnp.int32)], axis=1)
    nbr_packed = gather_tab(nbrtab, idx_in3)
    h_rows, item_rows = gather_small(emb, idx_in3, idx_item3)
    nbr_ids = nbr_packed[:, :S]
    nbr_w = lax.bitcast_convert_type(nbr_packed[:, S:2 * S], jnp.float32)
    idx3 = nbr_ids.T.reshape(NW, NCH_B, CH)
    nv_flat = gather_nbr(emb, idx3)
    nvt3 = nv_flat.reshape(S, N_IN, H)
    a4 = jnp.concatenate([a_0.T, a_1.T, a_2.T, a_3.T],
                         axis=0).astype(jnp.bfloat16)
    w1a = g_w1[:H].astype(jnp.bfloat16)
    w1l = g_w1[H:]
    w2r = g_w2.T
    w3 = g_w3.astype(jnp.bfloat16)
    bias = g_bias.reshape(1, H)
    maskf = mask_item.astype(jnp.float32)
    adj2 = adj.reshape(N_IN, L)
    loc_bf, sess_bf = _tc_local_call(h_rows, item_rows, maskf, adj2, a4)
    out = _tc_global_call(h_rows, sess_bf, loc_bf, nbr_w, nvt3,
                          w1a, w1l, w2r, w3, bias)
    return out.reshape(B, L, H)
